# baseline (speedup 1.0000x reference)
import functools

import jax
import jax.numpy as jnp
from jax import lax
from jax.experimental import pallas as pl
from jax.experimental.pallas import tpu as pltpu
from jax.experimental.pallas import tpu_sc as plsc

L = 16
DIM = 64
NUM_WORKERS = 32
CHUNK = 128
HALF = 256
_MARGIN = 1.0


def _rsqrt16(x):
    i = lax.bitcast_convert_type(x, jnp.int32)
    i = jnp.int32(0x5F3759DF) - lax.shift_right_arithmetic(i, 1)
    y = lax.bitcast_convert_type(i, jnp.float32)
    for _ in range(3):
        y = y * (1.5 - 0.5 * x * y * y)
    return y


def _make_kernel(rows_per_w, nchunk):
    mesh = plsc.VectorSubcoreMesh(core_axis_name="c", subcore_axis_name="s")

    @functools.partial(
        pl.kernel,
        mesh=mesh,
        compiler_params=pltpu.CompilerParams(
            needs_layout_passes=False, use_tc_tiling_on_sc=False),
        out_type=jax.ShapeDtypeStruct((NUM_WORKERS, L), jnp.float32),
        scratch_types=[
            pltpu.VMEM((rows_per_w, 3), jnp.int32),
            pltpu.VMEM((rows_per_w, 3), jnp.int32),
            pltpu.VMEM((nchunk, CHUNK), jnp.int32),
            pltpu.VMEM((nchunk, CHUNK), jnp.int32),
            pltpu.VMEM((nchunk, CHUNK), jnp.int32),
            pltpu.VMEM((nchunk, CHUNK), jnp.int32),
            pltpu.VMEM((nchunk, CHUNK), jnp.int32),
            pltpu.VMEM((nchunk, CHUNK), jnp.int32),
            pltpu.VMEM((HALF, DIM), jnp.float32),
            pltpu.VMEM((HALF, DIM), jnp.float32),
            pltpu.VMEM((HALF, DIM), jnp.float32),
            pltpu.VMEM((HALF, DIM), jnp.float32),
            pltpu.VMEM((HALF, DIM), jnp.float32),
            pltpu.VMEM((HALF, DIM), jnp.float32),
            pltpu.VMEM((rows_per_w,), jnp.float32),
            pltpu.VMEM((rows_per_w,), jnp.float32),
            pltpu.VMEM((L,), jnp.float32),
            pltpu.SemaphoreType.DMA,
            pltpu.SemaphoreType.DMA,
        ],
    )
    def transe_sc(pflat, nflat, ent, rel, out,
                  rawp, rawn,
                  phidx, pridx, ptidx, nhidx, nridx, ntidx,
                  hA, rA, tA, hB, rB, tB,
                  psc, nsc, pbuf, semA, semB):
        wid = lax.axis_index("s") * 2 + lax.axis_index("c")
        iota = lax.iota(jnp.int32, L)
        zf = jnp.zeros((L,), jnp.float32)
        zi = jnp.zeros((L,), jnp.int32)

        pltpu.sync_copy(pflat.at[pl.ds(wid * rows_per_w, rows_per_w)], rawp)
        pltpu.sync_copy(nflat.at[pl.ds(wid * rows_per_w, rows_per_w)], rawn)
        col0 = jnp.zeros((L,), jnp.int32)
        col1 = col0 + 1
        col2 = col0 + 2
        for raw, (hx, rx, tx) in ((rawp, (phidx, pridx, ptidx)),
                                  (rawn, (nhidx, nridx, ntidx))):
            for g in range(rows_per_w // L):
                rv = iota + g * L
                c, u = divmod(g, CHUNK // L)
                s = pl.ds(u * L, L)
                hx[c, s] = plsc.load_gather(raw, [rv, col0])
                rx[c, s] = plsc.load_gather(raw, [rv, col1])
                tx[c, s] = plsc.load_gather(raw, [rv, col2])

        def fire(hx, rx, tx, half, bufs, sem):
            cps = []
            for k in range(HALF // CHUNK):
                c = half * (HALF // CHUNK) + k
                d = pl.ds(k * CHUNK, CHUNK)
                cps.append(pltpu.async_copy(ent.at[hx.at[c]], bufs[0].at[d], sem))
                cps.append(pltpu.async_copy(rel.at[rx.at[c]], bufs[1].at[d], sem))
                cps.append(pltpu.async_copy(ent.at[tx.at[c]], bufs[2].at[d], sem))
            return cps

        def compute(bufs, scref, base):
            hrow, rrow, trow = bufs

            def blk(b, rowv):
                def p1(u, carry):
                    sh, st, colv = carry
                    for _ in range(16):
                        hv = plsc.load_gather(hrow, [rowv, colv])
                        tv = plsc.load_gather(trow, [rowv, colv])
                        sh = sh + hv * hv
                        st = st + tv * tv
                        colv = (colv + 1) & (DIM - 1)
                    return sh, st, colv

                sh, st, _ = lax.fori_loop(0, DIM // 16, p1, (zf, zf, iota))
                rih = _rsqrt16(sh)
                rit = _rsqrt16(st)

                def p2(u, carry):
                    acc, colv = carry
                    for _ in range(16):
                        hv = plsc.load_gather(hrow, [rowv, colv])
                        rv = plsc.load_gather(rrow, [rowv, colv])
                        tv = plsc.load_gather(trow, [rowv, colv])
                        acc = acc + jnp.abs(hv * rih + rv - tv * rit)
                        colv = (colv + 1) & (DIM - 1)
                    return acc, colv

                acc, _ = lax.fori_loop(0, DIM // 16, p2, (zf, iota))
                plsc.store_scatter(scref, [rowv + base], acc)
                return rowv + L

            lax.fori_loop(0, HALF // L, blk, iota)

        A = (hA, rA, tA)
        B = (hB, rB, tB)
        pending = [fire(phidx, pridx, ptidx, 0, A, semA),
                   fire(phidx, pridx, ptidx, 1, B, semB)]
        plan = [
            (A, psc, 0, (nhidx, nridx, ntidx, 0, A, semA)),
            (B, psc, HALF, (nhidx, nridx, ntidx, 1, B, semB)),
            (A, nsc, 0, None),
            (B, nsc, HALF, None),
        ]
        for bufs, scref, base, refire in plan:
            for c in pending.pop(0):
                c.wait()
            compute(bufs, scref, base)
            if refire is not None:
                pending.append(fire(*refire))

        accv = zf
        for b in range(rows_per_w // L):
            p = psc[pl.ds(b * L, L)]
            n = nsc[pl.ds(b * L, L)]
            accv = accv + jnp.maximum(p - n + _MARGIN, 0.0)
        pbuf[...] = accv
        pltpu.sync_copy(pbuf, out.at[wid])

    return transe_sc


def kernel(batch_positives, batch_negatives, entity_emb, relation_emb):
    batch = batch_positives.shape[0]
    rows_per_w = batch // NUM_WORKERS
    nchunk = rows_per_w // CHUNK

    partials = _make_kernel(rows_per_w, nchunk)(
        batch_positives, batch_negatives, entity_emb, relation_emb)
    return jnp.sum(partials) / jnp.float32(batch)


# speedup vs baseline: 1.1987x; 1.1987x over previous
import functools

import jax
import jax.numpy as jnp
from jax import lax
from jax.experimental import pallas as pl
from jax.experimental.pallas import tpu as pltpu
from jax.experimental.pallas import tpu_sc as plsc

L = 16
DIM = 64
NUM_WORKERS = 32
CHUNK = 128
HALF = 256
_MARGIN = 1.0


def _rsqrt16(x):
    i = lax.bitcast_convert_type(x, jnp.int32)
    i = jnp.int32(0x5F3759DF) - lax.shift_right_arithmetic(i, 1)
    y = lax.bitcast_convert_type(i, jnp.float32)
    for _ in range(3):
        y = y * (1.5 - 0.5 * x * y * y)
    return y


def _make_kernel(rows_per_w, nchunk):
    mesh = plsc.VectorSubcoreMesh(core_axis_name="c", subcore_axis_name="s")

    @functools.partial(
        pl.kernel,
        mesh=mesh,
        compiler_params=pltpu.CompilerParams(
            needs_layout_passes=False, use_tc_tiling_on_sc=False),
        out_type=jax.ShapeDtypeStruct((NUM_WORKERS, L), jnp.float32),
        scratch_types=[
            pltpu.VMEM((nchunk, CHUNK), jnp.int32),
            pltpu.VMEM((nchunk, CHUNK), jnp.int32),
            pltpu.VMEM((nchunk, CHUNK), jnp.int32),
            pltpu.VMEM((nchunk, CHUNK), jnp.int32),
            pltpu.VMEM((nchunk, CHUNK), jnp.int32),
            pltpu.VMEM((nchunk, CHUNK), jnp.int32),
            pltpu.VMEM((HALF, DIM), jnp.float32),
            pltpu.VMEM((HALF, DIM), jnp.float32),
            pltpu.VMEM((HALF, DIM), jnp.float32),
            pltpu.VMEM((HALF, DIM), jnp.float32),
            pltpu.VMEM((HALF, DIM), jnp.float32),
            pltpu.VMEM((HALF, DIM), jnp.float32),
            pltpu.VMEM((rows_per_w,), jnp.float32),
            pltpu.VMEM((rows_per_w,), jnp.float32),
            pltpu.VMEM((L,), jnp.float32),
            pltpu.SemaphoreType.DMA,
            pltpu.SemaphoreType.DMA,
        ],
    )
    def transe_sc(ph, pr, pt, nh, nr, nt, ent, rel, out,
                  phidx, pridx, ptidx, nhidx, nridx, ntidx,
                  hA, rA, tA, hB, rB, tB,
                  psc, nsc, pbuf, semA, semB):
        wid = lax.axis_index("s") * 2 + lax.axis_index("c")
        iota = lax.iota(jnp.int32, L)
        zf = jnp.zeros((L,), jnp.float32)
        zi = jnp.zeros((L,), jnp.int32)

        for src, dst in ((ph, phidx), (pr, pridx), (pt, ptidx),
                         (nh, nhidx), (nr, nridx), (nt, ntidx)):
            pltpu.sync_copy(src.at[wid], dst)

        def fire(hx, rx, tx, half, bufs, sem):
            cps = []
            for k in range(HALF // CHUNK):
                c = half * (HALF // CHUNK) + k
                d = pl.ds(k * CHUNK, CHUNK)
                cps.append(pltpu.async_copy(ent.at[hx.at[c]], bufs[0].at[d], sem))
                cps.append(pltpu.async_copy(rel.at[rx.at[c]], bufs[1].at[d], sem))
                cps.append(pltpu.async_copy(ent.at[tx.at[c]], bufs[2].at[d], sem))
            return cps

        def compute(bufs, scref, base):
            hrow, rrow, trow = bufs

            def blk(b, rowv):
                def p1(u, carry):
                    sh, st, colv = carry
                    for _ in range(16):
                        hv = plsc.load_gather(hrow, [rowv, colv])
                        tv = plsc.load_gather(trow, [rowv, colv])
                        sh = sh + hv * hv
                        st = st + tv * tv
                        colv = (colv + 1) & (DIM - 1)
                    return sh, st, colv

                sh, st, _ = lax.fori_loop(0, DIM // 16, p1, (zf, zf, iota))
                rih = _rsqrt16(sh)
                rit = _rsqrt16(st)

                def p2(u, carry):
                    acc, colv = carry
                    for _ in range(16):
                        hv = plsc.load_gather(hrow, [rowv, colv])
                        rv = plsc.load_gather(rrow, [rowv, colv])
                        tv = plsc.load_gather(trow, [rowv, colv])
                        acc = acc + jnp.abs(hv * rih + rv - tv * rit)
                        colv = (colv + 1) & (DIM - 1)
                    return acc, colv

                acc, _ = lax.fori_loop(0, DIM // 16, p2, (zf, iota))
                plsc.store_scatter(scref, [rowv + base], acc)
                return rowv + L

            lax.fori_loop(0, HALF // L, blk, iota)

        A = (hA, rA, tA)
        B = (hB, rB, tB)
        pending = [fire(phidx, pridx, ptidx, 0, A, semA),
                   fire(phidx, pridx, ptidx, 1, B, semB)]
        plan = [
            (A, psc, 0, (nhidx, nridx, ntidx, 0, A, semA)),
            (B, psc, HALF, (nhidx, nridx, ntidx, 1, B, semB)),
            (A, nsc, 0, None),
            (B, nsc, HALF, None),
        ]
        for bufs, scref, base, refire in plan:
            for c in pending.pop(0):
                c.wait()
            compute(bufs, scref, base)
            if refire is not None:
                pending.append(fire(*refire))

        accv = zf
        for b in range(rows_per_w // L):
            p = psc[pl.ds(b * L, L)]
            n = nsc[pl.ds(b * L, L)]
            accv = accv + jnp.maximum(p - n + _MARGIN, 0.0)
        pbuf[...] = accv
        pltpu.sync_copy(pbuf, out.at[wid])

    return transe_sc


def kernel(batch_positives, batch_negatives, entity_emb, relation_emb):
    batch = batch_positives.shape[0]
    rows_per_w = batch // NUM_WORKERS
    nchunk = rows_per_w // CHUNK

    def split(b):
        return (b[:, 0].reshape(NUM_WORKERS, nchunk, CHUNK),
                b[:, 1].reshape(NUM_WORKERS, nchunk, CHUNK),
                b[:, 2].reshape(NUM_WORKERS, nchunk, CHUNK))

    ph, pr, pt = split(batch_positives)
    nh, nr, nt = split(batch_negatives)
    partials = _make_kernel(rows_per_w, nchunk)(
        ph, pr, pt, nh, nr, nt, entity_emb, relation_emb)
    return jnp.sum(partials) / jnp.float32(batch)


# speedup vs baseline: 1.2059x; 1.0061x over previous
import functools

import jax
import jax.numpy as jnp
from jax import lax
from jax.experimental import pallas as pl
from jax.experimental.pallas import tpu as pltpu
from jax.experimental.pallas import tpu_sc as plsc

L = 16
DIM = 64
NUM_WORKERS = 32
CHUNK = 128
HALF = 256
_MARGIN = 1.0


def _rsqrt16(x):
    i = lax.bitcast_convert_type(x, jnp.int32)
    i = jnp.int32(0x5F3759DF) - lax.shift_right_arithmetic(i, 1)
    y = lax.bitcast_convert_type(i, jnp.float32)
    for _ in range(3):
        y = y * (1.5 - 0.5 * x * y * y)
    return y


def _make_kernel(rows_per_w, nchunk):
    mesh = plsc.VectorSubcoreMesh(core_axis_name="c", subcore_axis_name="s")

    @functools.partial(
        pl.kernel,
        mesh=mesh,
        compiler_params=pltpu.CompilerParams(
            needs_layout_passes=False, use_tc_tiling_on_sc=False),
        out_type=jax.ShapeDtypeStruct((NUM_WORKERS, L), jnp.float32),
        scratch_types=[
            pltpu.VMEM((rows_per_w,), jnp.int32),
            pltpu.VMEM((rows_per_w,), jnp.int32),
            pltpu.VMEM((rows_per_w,), jnp.int32),
            pltpu.VMEM((rows_per_w,), jnp.int32),
            pltpu.VMEM((rows_per_w,), jnp.int32),
            pltpu.VMEM((rows_per_w,), jnp.int32),
            pltpu.VMEM((HALF, DIM), jnp.float32),
            pltpu.VMEM((HALF, DIM), jnp.float32),
            pltpu.VMEM((HALF, DIM), jnp.float32),
            pltpu.VMEM((HALF, DIM), jnp.float32),
            pltpu.VMEM((HALF, DIM), jnp.float32),
            pltpu.VMEM((HALF, DIM), jnp.float32),
            pltpu.VMEM((rows_per_w,), jnp.float32),
            pltpu.VMEM((rows_per_w,), jnp.float32),
            pltpu.VMEM((L,), jnp.float32),
            pltpu.SemaphoreType.DMA,
            pltpu.SemaphoreType.DMA,
        ],
    )
    def transe_sc(ph, pr, pt, nh, nr, nt, ent, rel, out,
                  phidx, pridx, ptidx, nhidx, nridx, ntidx,
                  hA, rA, tA, hB, rB, tB,
                  psc, nsc, pbuf, semA, semB):
        wid = lax.axis_index("s") * 2 + lax.axis_index("c")
        iota = lax.iota(jnp.int32, L)
        zf = jnp.zeros((L,), jnp.float32)
        zi = jnp.zeros((L,), jnp.int32)

        for src, dst in ((ph, phidx), (pr, pridx), (pt, ptidx),
                         (nh, nhidx), (nr, nridx), (nt, ntidx)):
            pltpu.sync_copy(src.at[pl.ds(wid * rows_per_w, rows_per_w)], dst)

        def fire(hx, rx, tx, half, bufs, sem):
            cps = []
            for k in range(HALF // CHUNK):
                c = pl.ds((half * (HALF // CHUNK) + k) * CHUNK, CHUNK)
                d = pl.ds(k * CHUNK, CHUNK)
                cps.append(pltpu.async_copy(ent.at[hx.at[c]], bufs[0].at[d], sem))
                cps.append(pltpu.async_copy(rel.at[rx.at[c]], bufs[1].at[d], sem))
                cps.append(pltpu.async_copy(ent.at[tx.at[c]], bufs[2].at[d], sem))
            return cps

        def compute(bufs, scref, base):
            hrow, rrow, trow = bufs

            def blk(b, rowv):
                def p1(u, carry):
                    sh, st, colv = carry
                    for _ in range(16):
                        hv = plsc.load_gather(hrow, [rowv, colv])
                        tv = plsc.load_gather(trow, [rowv, colv])
                        sh = sh + hv * hv
                        st = st + tv * tv
                        colv = (colv + 1) & (DIM - 1)
                    return sh, st, colv

                sh, st, _ = lax.fori_loop(0, DIM // 16, p1, (zf, zf, iota))
                rih = _rsqrt16(sh)
                rit = _rsqrt16(st)

                def p2(u, carry):
                    acc, colv = carry
                    for _ in range(16):
                        hv = plsc.load_gather(hrow, [rowv, colv])
                        rv = plsc.load_gather(rrow, [rowv, colv])
                        tv = plsc.load_gather(trow, [rowv, colv])
                        acc = acc + jnp.abs(hv * rih + rv - tv * rit)
                        colv = (colv + 1) & (DIM - 1)
                    return acc, colv

                acc, _ = lax.fori_loop(0, DIM // 16, p2, (zf, iota))
                plsc.store_scatter(scref, [rowv + base], acc)
                return rowv + L

            lax.fori_loop(0, HALF // L, blk, iota)

        A = (hA, rA, tA)
        B = (hB, rB, tB)
        pending = [fire(phidx, pridx, ptidx, 0, A, semA),
                   fire(phidx, pridx, ptidx, 1, B, semB)]
        plan = [
            (A, psc, 0, (nhidx, nridx, ntidx, 0, A, semA)),
            (B, psc, HALF, (nhidx, nridx, ntidx, 1, B, semB)),
            (A, nsc, 0, None),
            (B, nsc, HALF, None),
        ]
        for bufs, scref, base, refire in plan:
            for c in pending.pop(0):
                c.wait()
            compute(bufs, scref, base)
            if refire is not None:
                pending.append(fire(*refire))

        accv = zf
        for b in range(rows_per_w // L):
            p = psc[pl.ds(b * L, L)]
            n = nsc[pl.ds(b * L, L)]
            accv = accv + jnp.maximum(p - n + _MARGIN, 0.0)
        pbuf[...] = accv
        pltpu.sync_copy(pbuf, out.at[wid])

    return transe_sc


def kernel(batch_positives, batch_negatives, entity_emb, relation_emb):
    batch = batch_positives.shape[0]
    rows_per_w = batch // NUM_WORKERS
    nchunk = rows_per_w // CHUNK

    def split(b):
        return b[:, 0], b[:, 1], b[:, 2]

    ph, pr, pt = split(batch_positives)
    nh, nr, nt = split(batch_negatives)
    partials = _make_kernel(rows_per_w, nchunk)(
        ph, pr, pt, nh, nr, nt, entity_emb, relation_emb)
    return jnp.sum(partials) / jnp.float32(batch)
